# Initial kernel scaffold; baseline (speedup 1.0000x reference)
import jax
import jax.numpy as jnp
from jax.experimental import pallas as pl


def kernel(x, W, b):
    raise NotImplementedError("write your pallas kernel here")


# speedup vs baseline: 1.1620x; 1.1620x over previous
import functools

import jax
import jax.numpy as jnp
from jax import lax
from jax.experimental import pallas as pl
from jax.experimental.pallas import tpu as pltpu
from jax.experimental.pallas import tpu_sc as plsc

N = 100000
D = 512
K = 25000

_NC, _NS, _L = 2, 16, 16
_NW = _NC * _NS
_CPL = 392
_TPC = _L * _CPL
_NP = _NS * _TPC
_NB = 256
_KPAD = 25088
_WOUT = _KPAD // _NW
_CH = 112
_NCH = _WOUT // _CH
_MSB = -2147483648

_BN = 2000


def _score_body(x_ref, w_ref, b_ref, o_ref):
    s = jnp.dot(x_ref[...], w_ref[...], preferred_element_type=jnp.float32)
    o_ref[...] = jax.nn.sigmoid(s + b_ref[0, 0])


def _scores(x, W, b):
    return pl.pallas_call(
        _score_body,
        grid=(N // _BN,),
        in_specs=[
            pl.BlockSpec((_BN, D), lambda i: (i, 0)),
            pl.BlockSpec((D, 1), lambda i: (0, 0)),
            pl.BlockSpec((1, 1), lambda i: (0, 0), memory_space=pltpu.SMEM),
        ],
        out_specs=pl.BlockSpec((_BN, 1), lambda i: (i, 0)),
        out_shape=jax.ShapeDtypeStruct((N, 1), jnp.float32),
    )(x, W, b.reshape(1, 1))[:, 0]


def _sc_body(x_hbm, sc_hbm, nv_hbm, idx_hbm,
             keys_t, pays_t, dest_t, ramp_t, sbuf, hist_l, lane_base, tile_h,
             cbase, carry_v, grid_v, histidx, outk, outp, widx, scv, rows_v,
             sk_a, sp_a, sk_b, sp_b, sgrid, sem1, sem2):
    cid = lax.axis_index("c")
    tid = lax.axis_index("s")
    wid = tid * _NC + cid
    iot = lax.iota(jnp.int32, _L)

    def mk_ramp(j, _):
        ramp_t[pl.ds(j * _L, _L)] = iot + j * _L + tid * _TPC
        return 0
    lax.fori_loop(0, _TPC // _L, mk_ramp, 0, unroll=False)

    def mk_hidx(j, _):
        histidx[pl.ds(j * _L, _L)] = iot + j * _L + tid * _NB
        return 0
    lax.fori_loop(0, _NB // _L, mk_hidx, 0, unroll=False)

    def mk_widx(j, _):
        widx[pl.ds(j * _L, _L)] = iot + j * _L + wid * _WOUT
        return 0
    lax.fori_loop(0, _WOUT // _L, mk_widx, 0, unroll=False)

    pltpu.sync_copy(sc_hbm.at[pl.ds(tid * _TPC, _TPC)], sbuf)

    def mk_keys(j, _):
        sl = pl.ds(j * _L, _L)
        bits = plsc.bitcast(sbuf[sl], jnp.int32)
        m = lax.shift_right_arithmetic(bits, 31)
        key = lax.bitwise_xor(bits, lax.bitwise_or(m, jnp.int32(_MSB)))
        k2 = lax.bitwise_not(key)
        e16 = ramp_t[sl]
        keys_t[sl] = jnp.where(e16 < N, k2, jnp.int32(-1))
        return 0
    lax.fori_loop(0, _TPC // _L, mk_keys, 0, unroll=False)

    c1 = pltpu.async_copy(keys_t, sk_a.at[ramp_t], sem1)
    c2 = pltpu.async_copy(ramp_t, sp_a.at[ramp_t], sem2)
    c1.wait()
    c2.wait()
    plsc.subcore_barrier()

    for p in range(4):
        sh = 8 * p
        src_k, src_p = (sk_a, sp_a) if p % 2 == 0 else (sk_b, sp_b)
        dst_k, dst_p = (sk_b, sp_b) if p % 2 == 0 else (sk_a, sp_a)

        c1 = pltpu.async_copy(src_k.at[ramp_t], keys_t, sem1)
        c2 = pltpu.async_copy(src_p.at[ramp_t], pays_t, sem2)
        c1.wait()
        c2.wait()

        def zero_h(j, _):
            hist_l[pl.ds(j * _L, _L)] = jnp.zeros((_L,), jnp.int32)
            return 0
        lax.fori_loop(0, (_L * _NB) // _L, zero_h, 0, unroll=False)

        def hstep(j, _):
            k16 = plsc.load_gather(keys_t, [iot * _CPL + j])
            d16 = lax.bitwise_and(lax.shift_right_logical(k16, sh), jnp.int32(255))
            plsc.addupdate_scatter(hist_l, [iot * _NB + d16], jnp.ones((_L,), jnp.int32))
            return 0
        lax.fori_loop(0, _CPL, hstep, 0, unroll=False)

        def treduce(d0, _):
            acc = jnp.zeros((_L,), jnp.int32)
            for l in range(_L):
                acc = acc + hist_l[pl.ds(l * _NB + d0 * _L, _L)]
            tile_h[pl.ds(d0 * _L, _L)] = acc
            return 0
        lax.fori_loop(0, _NB // _L, treduce, 0, unroll=False)

        pltpu.async_copy(tile_h, sgrid.at[histidx], sem1).wait()
        plsc.subcore_barrier()
        pltpu.sync_copy(sgrid, grid_v)

        carry_v[pl.ds(0, _L)] = jnp.zeros((_L,), jnp.int32)

        def dbase(d0, _):
            sl = pl.ds(d0 * _L, _L)
            tot = jnp.zeros((_L,), jnp.int32)
            pre = jnp.zeros((_L,), jnp.int32)
            for t in range(_NS):
                g = grid_v[pl.ds(t * _NB + d0 * _L, _L)]
                tot = tot + g
                pre = pre + jnp.where(t < tid, g, jnp.int32(0))
            incl = plsc.cumsum(tot)
            carry = carry_v[pl.ds(0, _L)]
            cbase[sl] = carry + (incl - tot) + pre
            carry_v[pl.ds(0, _L)] = carry + jnp.broadcast_to(incl[_L - 1], (_L,))
            return 0
        lax.fori_loop(0, _NB // _L, dbase, 0, unroll=False)

        def lbase(d0, _):
            sl = pl.ds(d0 * _L, _L)
            acc = cbase[sl]
            for l in range(_L):
                lane_base[pl.ds(l * _NB + d0 * _L, _L)] = acc
                acc = acc + hist_l[pl.ds(l * _NB + d0 * _L, _L)]
            return 0
        lax.fori_loop(0, _NB // _L, lbase, 0, unroll=False)

        def pstep(j, _):
            eaddr = iot * _CPL + j
            k16 = plsc.load_gather(keys_t, [eaddr])
            d16 = lax.bitwise_and(lax.shift_right_logical(k16, sh), jnp.int32(255))
            caddr = iot * _NB + d16
            dest = plsc.load_gather(lane_base, [caddr])
            plsc.store_scatter(lane_base, [caddr], dest + 1)
            plsc.store_scatter(dest_t, [eaddr], dest)
            return 0
        lax.fori_loop(0, _CPL, pstep, 0, unroll=False)

        c3 = pltpu.async_copy(keys_t, dst_k.at[dest_t], sem1)
        c4 = pltpu.async_copy(pays_t, dst_p.at[dest_t], sem2)
        c3.wait()
        c4.wait()
        plsc.subcore_barrier()

    c5 = pltpu.async_copy(sk_a.at[widx], outk, sem1)
    c6 = pltpu.async_copy(sp_a.at[widx], outp, sem2)
    c5.wait()
    c6.wait()

    pltpu.sync_copy(outp, idx_hbm.at[pl.ds(wid * _WOUT, _WOUT)])

    def mk_sc(j, _):
        kk = lax.bitwise_not(outk[pl.ds(j * _L, _L)])
        m = lax.shift_right_arithmetic(kk, 31)
        bits = jnp.where(m == jnp.int32(-1),
                         lax.bitwise_xor(kk, jnp.int32(_MSB)), lax.bitwise_not(kk))
        scv[pl.ds(j * _L, _L)] = plsc.bitcast(bits, jnp.float32)
        return 0
    lax.fori_loop(0, _WOUT // _L, mk_sc, 0, unroll=False)

    def chunk(c2, _):
        pltpu.async_copy(x_hbm.at[outp.at[pl.ds(c2 * _CH, _CH)]], rows_v, sem1).wait()

        def row(r, _):
            sv = jnp.broadcast_to(scv[pl.ds(c2 * _CH + r, _L)][0], (_L,))
            for col in range(D // _L):
                rows_v[r, pl.ds(col * _L, _L)] = rows_v[r, pl.ds(col * _L, _L)] * sv
            return 0
        lax.fori_loop(0, _CH, row, 0, unroll=False)
        pltpu.sync_copy(rows_v, nv_hbm.at[pl.ds(wid * _WOUT + c2 * _CH, _CH)])
        return 0
    lax.fori_loop(0, _NCH, chunk, 0, unroll=False)


def _sc_sort_gather(x, scores_pad):
    mesh = plsc.VectorSubcoreMesh(core_axis_name="c", subcore_axis_name="s")
    f = functools.partial(
        pl.kernel,
        mesh=mesh,
        compiler_params=pltpu.CompilerParams(needs_layout_passes=False),
        out_type=(jax.ShapeDtypeStruct((_KPAD, D), jnp.float32),
                  jax.ShapeDtypeStruct((_KPAD,), jnp.int32)),
        scratch_types=[
            pltpu.VMEM((_TPC,), jnp.int32),
            pltpu.VMEM((_TPC,), jnp.int32),
            pltpu.VMEM((_TPC,), jnp.int32),
            pltpu.VMEM((_TPC,), jnp.int32),
            pltpu.VMEM((_TPC,), jnp.float32),
            pltpu.VMEM((_L * _NB,), jnp.int32),
            pltpu.VMEM((_L * _NB,), jnp.int32),
            pltpu.VMEM((_NB,), jnp.int32),
            pltpu.VMEM((_NB,), jnp.int32),
            pltpu.VMEM((_L,), jnp.int32),
            pltpu.VMEM((_NS * _NB,), jnp.int32),
            pltpu.VMEM((_NB,), jnp.int32),
            pltpu.VMEM((_WOUT,), jnp.int32),
            pltpu.VMEM((_WOUT,), jnp.int32),
            pltpu.VMEM((_WOUT,), jnp.int32),
            pltpu.VMEM((_WOUT + _L,), jnp.float32),
            pltpu.VMEM((_CH, D), jnp.float32),
            pltpu.VMEM_SHARED((_NP,), jnp.int32),
            pltpu.VMEM_SHARED((_NP,), jnp.int32),
            pltpu.VMEM_SHARED((_NP,), jnp.int32),
            pltpu.VMEM_SHARED((_NP,), jnp.int32),
            pltpu.VMEM_SHARED((_NS * _NB,), jnp.int32),
            pltpu.SemaphoreType.DMA,
            pltpu.SemaphoreType.DMA,
        ],
    )(_sc_body)
    return f(x, scores_pad)


def kernel(x, W, b):
    scores = jax.nn.sigmoid(x @ W + b)[:, 0]
    scores_pad = jnp.concatenate([scores, jnp.zeros((_NP - N,), jnp.float32)])
    new_val, idx = _sc_sort_gather(x, scores_pad)
    return new_val[:K], idx[:K]


# speedup vs baseline: 1.3573x; 1.1681x over previous
import functools

import jax
import jax.numpy as jnp
from jax import lax
from jax.experimental import pallas as pl
from jax.experimental.pallas import tpu as pltpu
from jax.experimental.pallas import tpu_sc as plsc

N = 100000
D = 512
K = 25000

_NC, _NS, _L = 2, 16, 16
_NW = _NC * _NS
_CPL = 392
_TPC = _L * _CPL
_NP = _NS * _TPC
_NB = 256
_KPAD = 25088
_WOUT = _KPAD // _NW
_CH = 112
_NCH = _WOUT // _CH
_TAIL = K - (_KPAD - _CH)
_MSB = -2147483648

_BN = 2000


def _score_body(x_ref, w_ref, b_ref, o_ref):
    s = jnp.dot(x_ref[...], w_ref[...], preferred_element_type=jnp.float32)
    o_ref[...] = jax.nn.sigmoid(s + b_ref[0, 0])


def _scores(x, W, b):
    return pl.pallas_call(
        _score_body,
        grid=(N // _BN,),
        in_specs=[
            pl.BlockSpec((_BN, D), lambda i: (i, 0)),
            pl.BlockSpec((D, 1), lambda i: (0, 0)),
            pl.BlockSpec((1, 1), lambda i: (0, 0), memory_space=pltpu.SMEM),
        ],
        out_specs=pl.BlockSpec((_BN, 1), lambda i: (i, 0)),
        out_shape=jax.ShapeDtypeStruct((N, 1), jnp.float32),
    )(x, W, b.reshape(1, 1))[:, 0]


def _sc_body(x_hbm, sc_hbm, nv_hbm, idx_hbm,
             keys_t, pays_t, dest_t, ramp_t, sbuf, hist_l, lane_base, tile_h,
             cbase, carry_v, grid_v, histidx, outk, outp, widx, scv, rows_v,
             sk_a, sp_a, sk_b, sp_b, sgrid, sem1, sem2):
    cid = lax.axis_index("c")
    tid = lax.axis_index("s")
    wid = tid * _NC + cid
    iot = lax.iota(jnp.int32, _L)

    def mk_ramp(j, _):
        ramp_t[pl.ds(j * _L, _L)] = iot + j * _L + tid * _TPC
        return 0
    lax.fori_loop(0, _TPC // _L, mk_ramp, 0, unroll=4)

    def mk_hidx(j, _):
        histidx[pl.ds(j * _L, _L)] = iot + j * _L + tid * _NB
        return 0
    lax.fori_loop(0, _NB // _L, mk_hidx, 0, unroll=False)

    def mk_widx(j, _):
        widx[pl.ds(j * _L, _L)] = iot + j * _L + wid * _WOUT
        return 0
    lax.fori_loop(0, _WOUT // _L, mk_widx, 0, unroll=False)

    pltpu.sync_copy(sc_hbm.at[pl.ds(tid * _TPC, _TPC)], sbuf)

    def mk_keys(j, _):
        sl = pl.ds(j * _L, _L)
        bits = plsc.bitcast(sbuf[sl], jnp.int32)
        m = lax.shift_right_arithmetic(bits, 31)
        key = lax.bitwise_xor(bits, lax.bitwise_or(m, jnp.int32(_MSB)))
        k2 = lax.bitwise_not(key)
        e16 = ramp_t[sl]
        keys_t[sl] = jnp.where(e16 < N, k2, jnp.int32(-1))
        return 0
    lax.fori_loop(0, _TPC // _L, mk_keys, 0, unroll=4)

    for p in range(4):
        sh = 8 * p
        src_k, src_p = (sk_a, sp_a) if p % 2 == 0 else (sk_b, sp_b)
        dst_k, dst_p = (sk_b, sp_b) if p % 2 == 0 else (sk_a, sp_a)

        if p > 0:
            c1 = pltpu.async_copy(src_k.at[ramp_t], keys_t, sem1)
            c2 = pltpu.async_copy(src_p.at[ramp_t], pays_t, sem2)
            c1.wait()
            c2.wait()
        pays_src = ramp_t if p == 0 else pays_t

        def zero_h(j, _):
            hist_l[pl.ds(j * _L, _L)] = jnp.zeros((_L,), jnp.int32)
            return 0
        lax.fori_loop(0, (_L * _NB) // _L, zero_h, 0, unroll=False)

        def hstep(j, _):
            k16 = plsc.load_gather(keys_t, [iot * _CPL + j])
            d16 = lax.bitwise_and(lax.shift_right_logical(k16, sh), jnp.int32(255))
            plsc.addupdate_scatter(hist_l, [iot * _NB + d16], jnp.ones((_L,), jnp.int32))
            return 0
        lax.fori_loop(0, _CPL, hstep, 0, unroll=4)

        def treduce(d0, _):
            acc = jnp.zeros((_L,), jnp.int32)
            for l in range(_L):
                acc = acc + hist_l[pl.ds(l * _NB + d0 * _L, _L)]
            tile_h[pl.ds(d0 * _L, _L)] = acc
            return 0
        lax.fori_loop(0, _NB // _L, treduce, 0, unroll=False)

        pltpu.async_copy(tile_h, sgrid.at[histidx], sem1).wait()
        plsc.subcore_barrier()
        pltpu.sync_copy(sgrid, grid_v)

        carry_v[pl.ds(0, _L)] = jnp.zeros((_L,), jnp.int32)

        def dbase(d0, _):
            sl = pl.ds(d0 * _L, _L)
            tot = jnp.zeros((_L,), jnp.int32)
            pre = jnp.zeros((_L,), jnp.int32)
            for t in range(_NS):
                g = grid_v[pl.ds(t * _NB + d0 * _L, _L)]
                tot = tot + g
                pre = pre + jnp.where(t < tid, g, jnp.int32(0))
            incl = plsc.cumsum(tot)
            carry = carry_v[pl.ds(0, _L)]
            cbase[sl] = carry + (incl - tot) + pre
            carry_v[pl.ds(0, _L)] = carry + jnp.broadcast_to(incl[_L - 1], (_L,))
            return 0
        lax.fori_loop(0, _NB // _L, dbase, 0, unroll=False)

        def lbase(d0, _):
            sl = pl.ds(d0 * _L, _L)
            acc = cbase[sl]
            for l in range(_L):
                lane_base[pl.ds(l * _NB + d0 * _L, _L)] = acc
                acc = acc + hist_l[pl.ds(l * _NB + d0 * _L, _L)]
            return 0
        lax.fori_loop(0, _NB // _L, lbase, 0, unroll=False)

        def pstep(j, _):
            eaddr = iot * _CPL + j
            k16 = plsc.load_gather(keys_t, [eaddr])
            d16 = lax.bitwise_and(lax.shift_right_logical(k16, sh), jnp.int32(255))
            caddr = iot * _NB + d16
            dest = plsc.load_gather(lane_base, [caddr])
            plsc.store_scatter(lane_base, [caddr], dest + 1)
            plsc.store_scatter(dest_t, [eaddr], dest)
            return 0
        lax.fori_loop(0, _CPL, pstep, 0, unroll=2)

        c3 = pltpu.async_copy(keys_t, dst_k.at[dest_t], sem1)
        c4 = pltpu.async_copy(pays_src, dst_p.at[dest_t], sem2)
        c3.wait()
        c4.wait()
        plsc.subcore_barrier()

    c5 = pltpu.async_copy(sk_a.at[widx], outk, sem1)
    c6 = pltpu.async_copy(sp_a.at[widx], outp, sem2)
    c5.wait()
    c6.wait()

    @pl.when(wid < _NW - 1)
    def _():
        pltpu.sync_copy(outp, idx_hbm.at[pl.ds(wid * _WOUT, _WOUT)])

    @pl.when(wid == _NW - 1)
    def _():
        pltpu.sync_copy(outp.at[pl.ds(0, K - (_NW - 1) * _WOUT)],
                        idx_hbm.at[pl.ds(wid * _WOUT, K - (_NW - 1) * _WOUT)])

    def mk_sc(j, _):
        kk = lax.bitwise_not(outk[pl.ds(j * _L, _L)])
        m = lax.shift_right_arithmetic(kk, 31)
        bits = jnp.where(m == jnp.int32(-1),
                         lax.bitwise_xor(kk, jnp.int32(_MSB)), lax.bitwise_not(kk))
        scv[pl.ds(j * _L, _L)] = plsc.bitcast(bits, jnp.float32)
        return 0
    lax.fori_loop(0, _WOUT // _L, mk_sc, 0, unroll=False)

    def chunk(c2, _):
        pltpu.async_copy(x_hbm.at[outp.at[pl.ds(c2 * _CH, _CH)]], rows_v, sem1).wait()

        def row(r, _):
            sv = jnp.broadcast_to(scv[pl.ds(c2 * _CH + r, _L)][0], (_L,))
            for col in range(D // _L):
                rows_v[r, pl.ds(col * _L, _L)] = rows_v[r, pl.ds(col * _L, _L)] * sv
            return 0
        lax.fori_loop(0, _CH, row, 0, unroll=False)
        base_row = wid * _WOUT + c2 * _CH

        @pl.when(base_row + _CH <= K)
        def _():
            pltpu.sync_copy(rows_v, nv_hbm.at[pl.ds(base_row, _CH)])

        @pl.when(base_row + _CH > K)
        def _():
            pltpu.sync_copy(rows_v.at[pl.ds(0, _TAIL)], nv_hbm.at[pl.ds(base_row, _TAIL)])
        return 0
    lax.fori_loop(0, _NCH, chunk, 0, unroll=False)


def _sc_sort_gather(x, scores_pad):
    mesh = plsc.VectorSubcoreMesh(core_axis_name="c", subcore_axis_name="s")
    f = functools.partial(
        pl.kernel,
        mesh=mesh,
        compiler_params=pltpu.CompilerParams(needs_layout_passes=False),
        out_type=(jax.ShapeDtypeStruct((K, D), jnp.float32),
                  jax.ShapeDtypeStruct((K,), jnp.int32)),
        scratch_types=[
            pltpu.VMEM((_TPC,), jnp.int32),
            pltpu.VMEM((_TPC,), jnp.int32),
            pltpu.VMEM((_TPC,), jnp.int32),
            pltpu.VMEM((_TPC,), jnp.int32),
            pltpu.VMEM((_TPC,), jnp.float32),
            pltpu.VMEM((_L * _NB,), jnp.int32),
            pltpu.VMEM((_L * _NB,), jnp.int32),
            pltpu.VMEM((_NB,), jnp.int32),
            pltpu.VMEM((_NB,), jnp.int32),
            pltpu.VMEM((_L,), jnp.int32),
            pltpu.VMEM((_NS * _NB,), jnp.int32),
            pltpu.VMEM((_NB,), jnp.int32),
            pltpu.VMEM((_WOUT,), jnp.int32),
            pltpu.VMEM((_WOUT,), jnp.int32),
            pltpu.VMEM((_WOUT,), jnp.int32),
            pltpu.VMEM((_WOUT + _L,), jnp.float32),
            pltpu.VMEM((_CH, D), jnp.float32),
            pltpu.VMEM_SHARED((_NP,), jnp.int32),
            pltpu.VMEM_SHARED((_NP,), jnp.int32),
            pltpu.VMEM_SHARED((_NP,), jnp.int32),
            pltpu.VMEM_SHARED((_NP,), jnp.int32),
            pltpu.VMEM_SHARED((_NS * _NB,), jnp.int32),
            pltpu.SemaphoreType.DMA,
            pltpu.SemaphoreType.DMA,
        ],
    )(_sc_body)
    return f(x, scores_pad)


def kernel(x, W, b):
    scores = jax.nn.sigmoid(x @ W + b)[:, 0]
    scores_pad = jnp.concatenate([scores, jnp.zeros((_NP - N,), jnp.float32)])
    new_val, idx = _sc_sort_gather(x, scores_pad)
    return new_val, idx


# speedup vs baseline: 1.3784x; 1.0155x over previous
import functools

import jax
import jax.numpy as jnp
from jax import lax
from jax.experimental import pallas as pl
from jax.experimental.pallas import tpu as pltpu
from jax.experimental.pallas import tpu_sc as plsc

N = 100000
D = 512
K = 25000

_NC, _NS, _L = 2, 16, 16
_NW = _NC * _NS
_CPL = 392
_TPC = _L * _CPL
_NP = _NS * _TPC
_NB = 256
_KPAD = 25088
_WOUT = _KPAD // _NW
_CH = 56
_NCH = _WOUT // _CH
_TAIL = 24
_MSB = -2147483648

_BN = 2000


def _score_body(x_ref, w_ref, b_ref, o_ref):
    s = jnp.dot(x_ref[...], w_ref[...], preferred_element_type=jnp.float32)
    o_ref[...] = jax.nn.sigmoid(s + b_ref[0, 0])


def _scores(x, W, b):
    return pl.pallas_call(
        _score_body,
        grid=(N // _BN,),
        in_specs=[
            pl.BlockSpec((_BN, D), lambda i: (i, 0)),
            pl.BlockSpec((D, 1), lambda i: (0, 0)),
            pl.BlockSpec((1, 1), lambda i: (0, 0), memory_space=pltpu.SMEM),
        ],
        out_specs=pl.BlockSpec((_BN, 1), lambda i: (i, 0)),
        out_shape=jax.ShapeDtypeStruct((N, 1), jnp.float32),
    )(x, W, b.reshape(1, 1))[:, 0]


def _sc_body(x_hbm, sc_hbm, nv_hbm, idx_hbm,
             keys_t, pays_t, dest_t, ramp_t, sbuf, hist_l, lane_base, tile_h,
             cbase, carry_v, grid_v, histidx, outk, outp, widx, scv, rows_v,
             sk_a, sp_a, sk_b, sp_b, sgrid, sem1, sem2):
    cid = lax.axis_index("c")
    tid = lax.axis_index("s")
    wid = tid * _NC + cid
    iot = lax.iota(jnp.int32, _L)

    def mk_ramp(j, _):
        ramp_t[pl.ds(j * _L, _L)] = iot + j * _L + tid * _TPC
        return 0
    lax.fori_loop(0, _TPC // _L, mk_ramp, 0, unroll=4)

    def mk_hidx(j, _):
        histidx[pl.ds(j * _L, _L)] = iot + j * _L + tid * _NB
        return 0
    lax.fori_loop(0, _NB // _L, mk_hidx, 0, unroll=False)

    def mk_widx(j, _):
        widx[pl.ds(j * _L, _L)] = iot + j * _L + wid * _WOUT
        return 0
    lax.fori_loop(0, _WOUT // _L, mk_widx, 0, unroll=False)

    pltpu.sync_copy(sc_hbm.at[pl.ds(tid * _TPC, _TPC)], sbuf)

    def mk_keys(j, _):
        sl = pl.ds(j * _L, _L)
        bits = plsc.bitcast(sbuf[sl], jnp.int32)
        m = lax.shift_right_arithmetic(bits, 31)
        key = lax.bitwise_xor(bits, lax.bitwise_or(m, jnp.int32(_MSB)))
        k2 = lax.bitwise_not(key)
        e16 = ramp_t[sl]
        keys_t[sl] = jnp.where(e16 < N, k2, jnp.int32(-1))
        return 0
    lax.fori_loop(0, _TPC // _L, mk_keys, 0, unroll=4)

    for p in range(4):
        sh = 8 * p
        src_k, src_p = (sk_a, sp_a) if p % 2 == 0 else (sk_b, sp_b)
        dst_k, dst_p = (sk_b, sp_b) if p % 2 == 0 else (sk_a, sp_a)

        if p > 0:
            c1 = pltpu.async_copy(src_k.at[ramp_t], keys_t, sem1)
            c2 = pltpu.async_copy(src_p.at[ramp_t], pays_t, sem2)
            c1.wait()
            c2.wait()
        pays_src = ramp_t if p == 0 else pays_t

        def zero_h(j, _):
            hist_l[pl.ds(j * _L, _L)] = jnp.zeros((_L,), jnp.int32)
            return 0
        lax.fori_loop(0, (_L * _NB) // _L, zero_h, 0, unroll=False)

        def hstep(j, _):
            k16 = plsc.load_gather(keys_t, [iot * _CPL + j])
            d16 = lax.bitwise_and(lax.shift_right_logical(k16, sh), jnp.int32(255))
            plsc.addupdate_scatter(hist_l, [iot * _NB + d16], jnp.ones((_L,), jnp.int32))
            return 0
        lax.fori_loop(0, _CPL, hstep, 0, unroll=4)

        def treduce(d0, _):
            acc = jnp.zeros((_L,), jnp.int32)
            for l in range(_L):
                acc = acc + hist_l[pl.ds(l * _NB + d0 * _L, _L)]
            tile_h[pl.ds(d0 * _L, _L)] = acc
            return 0
        lax.fori_loop(0, _NB // _L, treduce, 0, unroll=False)

        pltpu.async_copy(tile_h, sgrid.at[histidx], sem1).wait()
        plsc.subcore_barrier()
        pltpu.sync_copy(sgrid, grid_v)

        carry_v[pl.ds(0, _L)] = jnp.zeros((_L,), jnp.int32)

        def dbase(d0, _):
            sl = pl.ds(d0 * _L, _L)
            tot = jnp.zeros((_L,), jnp.int32)
            pre = jnp.zeros((_L,), jnp.int32)
            for t in range(_NS):
                g = grid_v[pl.ds(t * _NB + d0 * _L, _L)]
                tot = tot + g
                pre = pre + jnp.where(t < tid, g, jnp.int32(0))
            incl = plsc.cumsum(tot)
            carry = carry_v[pl.ds(0, _L)]
            cbase[sl] = carry + (incl - tot) + pre
            carry_v[pl.ds(0, _L)] = carry + jnp.broadcast_to(incl[_L - 1], (_L,))
            return 0
        lax.fori_loop(0, _NB // _L, dbase, 0, unroll=False)

        def lbase(d0, _):
            sl = pl.ds(d0 * _L, _L)
            acc = cbase[sl]
            for l in range(_L):
                lane_base[pl.ds(l * _NB + d0 * _L, _L)] = acc
                acc = acc + hist_l[pl.ds(l * _NB + d0 * _L, _L)]
            return 0
        lax.fori_loop(0, _NB // _L, lbase, 0, unroll=False)

        def pstep(j, _):
            eaddr = iot * _CPL + j
            k16 = plsc.load_gather(keys_t, [eaddr])
            d16 = lax.bitwise_and(lax.shift_right_logical(k16, sh), jnp.int32(255))
            caddr = iot * _NB + d16
            dest = plsc.load_gather(lane_base, [caddr])
            plsc.store_scatter(lane_base, [caddr], dest + 1)
            plsc.store_scatter(dest_t, [eaddr], dest)
            return 0
        lax.fori_loop(0, _CPL, pstep, 0, unroll=4)

        c3 = pltpu.async_copy(keys_t, dst_k.at[dest_t], sem1)
        c4 = pltpu.async_copy(pays_src, dst_p.at[dest_t], sem2)
        c3.wait()
        c4.wait()
        plsc.subcore_barrier()

    c5 = pltpu.async_copy(sk_a.at[widx], outk, sem1)
    c6 = pltpu.async_copy(sp_a.at[widx], outp, sem2)
    c5.wait()
    c6.wait()

    @pl.when(wid < _NW - 1)
    def _():
        pltpu.sync_copy(outp, idx_hbm.at[pl.ds(wid * _WOUT, _WOUT)])

    @pl.when(wid == _NW - 1)
    def _():
        pltpu.sync_copy(outp.at[pl.ds(0, K - (_NW - 1) * _WOUT)],
                        idx_hbm.at[pl.ds(wid * _WOUT, K - (_NW - 1) * _WOUT)])

    def mk_sc(j, _):
        kk = lax.bitwise_not(outk[pl.ds(j * _L, _L)])
        m = lax.shift_right_arithmetic(kk, 31)
        bits = jnp.where(m == jnp.int32(-1),
                         lax.bitwise_xor(kk, jnp.int32(_MSB)), lax.bitwise_not(kk))
        scv[pl.ds(j * _L, _L)] = plsc.bitcast(bits, jnp.float32)
        return 0
    lax.fori_loop(0, _WOUT // _L, mk_sc, 0, unroll=False)

    g1 = pltpu.async_copy(x_hbm.at[outp.at[pl.ds(0, _CH)]], rows_v.at[0], sem1)
    g2 = pltpu.async_copy(x_hbm.at[outp.at[pl.ds(_CH, _CH)]], rows_v.at[1], sem2)

    def chunk(j, _):
        for par in range(2):
            c2 = j * 2 + par
            sem = sem1 if par == 0 else sem2
            pltpu.make_async_copy(
                x_hbm.at[outp.at[pl.ds(c2 * _CH, _CH)]], rows_v.at[par], sem).wait()

            def row(r, _):
                sv = jnp.broadcast_to(scv[pl.ds(c2 * _CH + r, _L)][0], (_L,))
                for col in range(D // _L):
                    rows_v[par, r, pl.ds(col * _L, _L)] = (
                        rows_v[par, r, pl.ds(col * _L, _L)] * sv)
                return 0
            lax.fori_loop(0, _CH, row, 0, unroll=2)
            base_row = wid * _WOUT + c2 * _CH

            @pl.when(base_row + _CH <= K)
            def _():
                pltpu.sync_copy(rows_v.at[par], nv_hbm.at[pl.ds(base_row, _CH)])

            @pl.when(jnp.logical_and(base_row < K, base_row + _CH > K))
            def _():
                pltpu.sync_copy(rows_v.at[par, pl.ds(0, _TAIL)],
                                nv_hbm.at[pl.ds(base_row, _TAIL)])

            @pl.when(j < (_NCH - 2) // 2)
            def _():
                pltpu.async_copy(
                    x_hbm.at[outp.at[pl.ds((c2 + 2) * _CH, _CH)]], rows_v.at[par], sem)
        return 0

    lax.fori_loop(0, _NCH // 2, chunk, 0, unroll=False)

def _sc_sort_gather(x, scores_pad):
    mesh = plsc.VectorSubcoreMesh(core_axis_name="c", subcore_axis_name="s")
    f = functools.partial(
        pl.kernel,
        mesh=mesh,
        compiler_params=pltpu.CompilerParams(needs_layout_passes=False),
        out_type=(jax.ShapeDtypeStruct((K, D), jnp.float32),
                  jax.ShapeDtypeStruct((K,), jnp.int32)),
        scratch_types=[
            pltpu.VMEM((_TPC,), jnp.int32),
            pltpu.VMEM((_TPC,), jnp.int32),
            pltpu.VMEM((_TPC,), jnp.int32),
            pltpu.VMEM((_TPC,), jnp.int32),
            pltpu.VMEM((_TPC,), jnp.float32),
            pltpu.VMEM((_L * _NB,), jnp.int32),
            pltpu.VMEM((_L * _NB,), jnp.int32),
            pltpu.VMEM((_NB,), jnp.int32),
            pltpu.VMEM((_NB,), jnp.int32),
            pltpu.VMEM((_L,), jnp.int32),
            pltpu.VMEM((_NS * _NB,), jnp.int32),
            pltpu.VMEM((_NB,), jnp.int32),
            pltpu.VMEM((_WOUT,), jnp.int32),
            pltpu.VMEM((_WOUT,), jnp.int32),
            pltpu.VMEM((_WOUT,), jnp.int32),
            pltpu.VMEM((_WOUT + _L,), jnp.float32),
            pltpu.VMEM((2, _CH, D), jnp.float32),
            pltpu.VMEM_SHARED((_NP,), jnp.int32),
            pltpu.VMEM_SHARED((_NP,), jnp.int32),
            pltpu.VMEM_SHARED((_NP,), jnp.int32),
            pltpu.VMEM_SHARED((_NP,), jnp.int32),
            pltpu.VMEM_SHARED((_NS * _NB,), jnp.int32),
            pltpu.SemaphoreType.DMA,
            pltpu.SemaphoreType.DMA,
        ],
    )(_sc_body)
    return f(x, scores_pad)


def kernel(x, W, b):
    scores = jax.nn.sigmoid(x @ W + b)[:, 0]
    scores_pad = jnp.concatenate([scores, jnp.zeros((_NP - N,), jnp.float32)])
    new_val, idx = _sc_sort_gather(x, scores_pad)
    return new_val, idx


# speedup vs baseline: 1.4177x; 1.0286x over previous
import functools

import jax
import jax.numpy as jnp
from jax import lax
from jax.experimental import pallas as pl
from jax.experimental.pallas import tpu as pltpu
from jax.experimental.pallas import tpu_sc as plsc

N = 100000
D = 512
K = 25000

_NC, _NS, _L = 2, 16, 16
_NW = _NC * _NS
_CPL = 392
_TPC = _L * _CPL
_NP = _NS * _TPC
_NB = 256
_KPAD = 25088
_WOUT = _KPAD // _NW
_CH = 56
_NCH = _WOUT // _CH
_TAIL = 24
_MSB = -2147483648

_BN = 2000


def _score_body(x_ref, w_ref, b_ref, o_ref):
    s = jnp.dot(x_ref[...], w_ref[...], preferred_element_type=jnp.float32)
    o_ref[...] = jax.nn.sigmoid(s + b_ref[0, 0])


def _scores(x, W, b):
    return pl.pallas_call(
        _score_body,
        grid=(N // _BN,),
        in_specs=[
            pl.BlockSpec((_BN, D), lambda i: (i, 0)),
            pl.BlockSpec((D, 1), lambda i: (0, 0)),
            pl.BlockSpec((1, 1), lambda i: (0, 0), memory_space=pltpu.SMEM),
        ],
        out_specs=pl.BlockSpec((_BN, 1), lambda i: (i, 0)),
        out_shape=jax.ShapeDtypeStruct((N, 1), jnp.float32),
    )(x, W, b.reshape(1, 1))[:, 0]


def _sc_body(x_hbm, sc_hbm, nv_hbm, idx_hbm,
             keys_t, pays_t, dest_t, ramp_t, sbuf, hist_l, lane_base, tile_h,
             cbase, carry_v, grid_v, histidx, outk, outp, widx, scv, rows_v,
             sk_a, sp_a, sk_b, sp_b, sgrid, sem1, sem2):
    cid = lax.axis_index("c")
    tid = lax.axis_index("s")
    wid = tid * _NC + cid
    iot = lax.iota(jnp.int32, _L)

    def mk_ramp(j, _):
        ramp_t[pl.ds(j * _L, _L)] = iot + j * _L + tid * _TPC
        return 0
    lax.fori_loop(0, _TPC // _L, mk_ramp, 0, unroll=4)

    def mk_hidx(j, _):
        histidx[pl.ds(j * _L, _L)] = iot + j * _L + tid * _NB
        return 0
    lax.fori_loop(0, _NB // _L, mk_hidx, 0, unroll=4)

    def mk_widx(j, _):
        widx[pl.ds(j * _L, _L)] = iot + j * _L + wid * _WOUT
        return 0
    lax.fori_loop(0, _WOUT // _L, mk_widx, 0, unroll=4)

    pltpu.sync_copy(sc_hbm.at[pl.ds(tid * _TPC, _TPC)], sbuf)

    def mk_keys(j, _):
        sl = pl.ds(j * _L, _L)
        bits = plsc.bitcast(sbuf[sl], jnp.int32)
        m = lax.shift_right_arithmetic(bits, 31)
        key = lax.bitwise_xor(bits, lax.bitwise_or(m, jnp.int32(_MSB)))
        k2 = lax.bitwise_not(key)
        e16 = ramp_t[sl]
        keys_t[sl] = jnp.where(e16 < N, k2, jnp.int32(-1))
        return 0
    lax.fori_loop(0, _TPC // _L, mk_keys, 0, unroll=4)

    for p in range(4):
        sh = 8 * p
        src_k, src_p = (sk_a, sp_a) if p % 2 == 0 else (sk_b, sp_b)
        dst_k, dst_p = (sk_b, sp_b) if p % 2 == 0 else (sk_a, sp_a)

        if p > 0:
            c1 = pltpu.async_copy(src_k.at[ramp_t], keys_t, sem1)
            c2 = pltpu.async_copy(src_p.at[ramp_t], pays_t, sem2)
            c1.wait()
            c2.wait()
        pays_src = ramp_t if p == 0 else pays_t

        def zero_h(j, _):
            hist_l[pl.ds(j * _L, _L)] = jnp.zeros((_L,), jnp.int32)
            return 0
        lax.fori_loop(0, (_L * _NB) // _L, zero_h, 0, unroll=8)

        def hstep(j, _):
            k16 = plsc.load_gather(keys_t, [iot * _CPL + j])
            d16 = lax.bitwise_and(lax.shift_right_logical(k16, sh), jnp.int32(255))
            plsc.addupdate_scatter(hist_l, [iot * _NB + d16], jnp.ones((_L,), jnp.int32))
            return 0
        lax.fori_loop(0, _CPL, hstep, 0, unroll=8)

        def treduce(d0, _):
            acc = jnp.zeros((_L,), jnp.int32)
            for l in range(_L):
                acc = acc + hist_l[pl.ds(l * _NB + d0 * _L, _L)]
            tile_h[pl.ds(d0 * _L, _L)] = acc
            return 0
        lax.fori_loop(0, _NB // _L, treduce, 0, unroll=2)

        pltpu.async_copy(tile_h, sgrid.at[histidx], sem1).wait()
        plsc.subcore_barrier()
        pltpu.sync_copy(sgrid, grid_v)

        carry_v[pl.ds(0, _L)] = jnp.zeros((_L,), jnp.int32)

        def dbase(d0, _):
            sl = pl.ds(d0 * _L, _L)
            tot = jnp.zeros((_L,), jnp.int32)
            pre = jnp.zeros((_L,), jnp.int32)
            for t in range(_NS):
                g = grid_v[pl.ds(t * _NB + d0 * _L, _L)]
                tot = tot + g
                pre = pre + jnp.where(t < tid, g, jnp.int32(0))
            incl = plsc.cumsum(tot)
            carry = carry_v[pl.ds(0, _L)]
            cbase[sl] = carry + (incl - tot) + pre
            carry_v[pl.ds(0, _L)] = carry + jnp.broadcast_to(incl[_L - 1], (_L,))
            return 0
        lax.fori_loop(0, _NB // _L, dbase, 0, unroll=False)

        def lbase(d0, _):
            sl = pl.ds(d0 * _L, _L)
            acc = cbase[sl]
            for l in range(_L):
                lane_base[pl.ds(l * _NB + d0 * _L, _L)] = acc
                acc = acc + hist_l[pl.ds(l * _NB + d0 * _L, _L)]
            return 0
        lax.fori_loop(0, _NB // _L, lbase, 0, unroll=2)

        def pstep(j, _):
            eaddr = iot * _CPL + j
            k16 = plsc.load_gather(keys_t, [eaddr])
            d16 = lax.bitwise_and(lax.shift_right_logical(k16, sh), jnp.int32(255))
            caddr = iot * _NB + d16
            dest = plsc.load_gather(lane_base, [caddr])
            plsc.store_scatter(lane_base, [caddr], dest + 1)
            plsc.store_scatter(dest_t, [eaddr], dest)
            return 0
        lax.fori_loop(0, _CPL, pstep, 0, unroll=4)

        c3 = pltpu.async_copy(keys_t, dst_k.at[dest_t], sem1)
        c4 = pltpu.async_copy(pays_src, dst_p.at[dest_t], sem2)
        c3.wait()
        c4.wait()
        plsc.subcore_barrier()

    c5 = pltpu.async_copy(sk_a.at[widx], outk, sem1)
    c6 = pltpu.async_copy(sp_a.at[widx], outp, sem2)
    c5.wait()
    c6.wait()

    @pl.when(wid < _NW - 1)
    def _():
        pltpu.sync_copy(outp, idx_hbm.at[pl.ds(wid * _WOUT, _WOUT)])

    @pl.when(wid == _NW - 1)
    def _():
        pltpu.sync_copy(outp.at[pl.ds(0, K - (_NW - 1) * _WOUT)],
                        idx_hbm.at[pl.ds(wid * _WOUT, K - (_NW - 1) * _WOUT)])

    def mk_sc(j, _):
        kk = lax.bitwise_not(outk[pl.ds(j * _L, _L)])
        m = lax.shift_right_arithmetic(kk, 31)
        bits = jnp.where(m == jnp.int32(-1),
                         lax.bitwise_xor(kk, jnp.int32(_MSB)), lax.bitwise_not(kk))
        scv[pl.ds(j * _L, _L)] = plsc.bitcast(bits, jnp.float32)
        return 0
    lax.fori_loop(0, _WOUT // _L, mk_sc, 0, unroll=4)

    g1 = pltpu.async_copy(x_hbm.at[outp.at[pl.ds(0, _CH)]], rows_v.at[0], sem1)
    g2 = pltpu.async_copy(x_hbm.at[outp.at[pl.ds(_CH, _CH)]], rows_v.at[1], sem2)

    def chunk(j, _):
        for par in range(2):
            c2 = j * 2 + par
            sem = sem1 if par == 0 else sem2
            pltpu.make_async_copy(
                x_hbm.at[outp.at[pl.ds(c2 * _CH, _CH)]], rows_v.at[par], sem).wait()

            def row(r, _):
                sv = jnp.broadcast_to(scv[pl.ds(c2 * _CH + r, _L)][0], (_L,))
                for col in range(D // _L):
                    rows_v[par, r, pl.ds(col * _L, _L)] = (
                        rows_v[par, r, pl.ds(col * _L, _L)] * sv)
                return 0
            lax.fori_loop(0, _CH, row, 0, unroll=2)
            base_row = wid * _WOUT + c2 * _CH

            @pl.when(base_row + _CH <= K)
            def _():
                pltpu.sync_copy(rows_v.at[par], nv_hbm.at[pl.ds(base_row, _CH)])

            @pl.when(jnp.logical_and(base_row < K, base_row + _CH > K))
            def _():
                pltpu.sync_copy(rows_v.at[par, pl.ds(0, _TAIL)],
                                nv_hbm.at[pl.ds(base_row, _TAIL)])

            @pl.when(j < (_NCH - 2) // 2)
            def _():
                pltpu.async_copy(
                    x_hbm.at[outp.at[pl.ds((c2 + 2) * _CH, _CH)]], rows_v.at[par], sem)
        return 0

    lax.fori_loop(0, _NCH // 2, chunk, 0, unroll=False)

def _sc_sort_gather(x, scores_pad):
    mesh = plsc.VectorSubcoreMesh(core_axis_name="c", subcore_axis_name="s")
    f = functools.partial(
        pl.kernel,
        mesh=mesh,
        compiler_params=pltpu.CompilerParams(needs_layout_passes=False),
        out_type=(jax.ShapeDtypeStruct((K, D), jnp.float32),
                  jax.ShapeDtypeStruct((K,), jnp.int32)),
        scratch_types=[
            pltpu.VMEM((_TPC,), jnp.int32),
            pltpu.VMEM((_TPC,), jnp.int32),
            pltpu.VMEM((_TPC,), jnp.int32),
            pltpu.VMEM((_TPC,), jnp.int32),
            pltpu.VMEM((_TPC,), jnp.float32),
            pltpu.VMEM((_L * _NB,), jnp.int32),
            pltpu.VMEM((_L * _NB,), jnp.int32),
            pltpu.VMEM((_NB,), jnp.int32),
            pltpu.VMEM((_NB,), jnp.int32),
            pltpu.VMEM((_L,), jnp.int32),
            pltpu.VMEM((_NS * _NB,), jnp.int32),
            pltpu.VMEM((_NB,), jnp.int32),
            pltpu.VMEM((_WOUT,), jnp.int32),
            pltpu.VMEM((_WOUT,), jnp.int32),
            pltpu.VMEM((_WOUT,), jnp.int32),
            pltpu.VMEM((_WOUT + _L,), jnp.float32),
            pltpu.VMEM((2, _CH, D), jnp.float32),
            pltpu.VMEM_SHARED((_NP,), jnp.int32),
            pltpu.VMEM_SHARED((_NP,), jnp.int32),
            pltpu.VMEM_SHARED((_NP,), jnp.int32),
            pltpu.VMEM_SHARED((_NP,), jnp.int32),
            pltpu.VMEM_SHARED((_NS * _NB,), jnp.int32),
            pltpu.SemaphoreType.DMA,
            pltpu.SemaphoreType.DMA,
        ],
    )(_sc_body)
    return f(x, scores_pad)


def kernel(x, W, b):
    scores = jax.nn.sigmoid(x @ W + b)[:, 0]
    scores_pad = jnp.concatenate([scores, jnp.zeros((_NP - N,), jnp.float32)])
    new_val, idx = _sc_sort_gather(x, scores_pad)
    return new_val, idx
